# baseline (speedup 1.0000x reference)
import functools

import jax
import jax.numpy as jnp
from jax import lax
from jax.experimental import pallas as pl
from jax.experimental.pallas import tpu as pltpu
from jax.experimental.pallas import tpu_sc as plsc


def _jet_proj(x_jet, W_jet):
    Bv = x_jet.shape[0]
    EMBv = W_jet.shape[0]

    def body(xj_ref, w_ref, out_ref):
        out_ref[:] = lax.dot_general(
            xj_ref[:], w_ref[:],
            dimension_numbers=(((1,), (1,)), ((), ())),
            preferred_element_type=jnp.float32)

    return pl.pallas_call(
        body,
        out_shape=jax.ShapeDtypeStruct((Bv, EMBv), jnp.float32),
    )(x_jet, W_jet)


def _embed_add(x, jet, table):
    Bv, Lv = x.shape
    EMBv = table.shape[1]
    LANES = 16
    nblk = EMBv // LANES

    mesh = plsc.VectorSubcoreMesh(core_axis_name="c", subcore_axis_name="s")
    NC, NS = mesh.num_cores, mesh.num_subcores
    NW = NC * NS
    nb = Bv // NW

    L0 = min(128, Lv)
    L1 = Lv - L0

    @functools.partial(
        pl.kernel,
        out_type=jax.ShapeDtypeStruct((Bv, Lv, EMBv), jnp.float32),
        mesh=mesh,
        scratch_types=[
            pltpu.VMEM((Lv,), jnp.int32),
            pltpu.VMEM((EMBv,), jnp.float32),
            pltpu.VMEM((Lv, EMBv), jnp.float32),
            pltpu.SemaphoreType.DMA,
        ],
        compiler_params=pltpu.CompilerParams(use_tc_tiling_on_sc=False),
    )
    def sc_k(x_hbm, jet_hbm, table_hbm, out_hbm, idx_v, jet_v, buf_v, sem):
        wid = lax.axis_index("s") * NC + lax.axis_index("c")
        base_b = wid * nb

        def body(j, carry):
            gb = base_b + j
            pltpu.sync_copy(x_hbm.at[gb], idx_v)
            pltpu.sync_copy(jet_hbm.at[gb], jet_v)
            c1 = pltpu.async_copy(
                table_hbm.at[idx_v.at[pl.ds(0, L0)]],
                buf_v.at[pl.ds(0, L0)], sem)
            if L1 > 0:
                c2 = pltpu.async_copy(
                    table_hbm.at[idx_v.at[pl.ds(L0, L1)]],
                    buf_v.at[pl.ds(L0, L1)], sem)
            c1.wait()
            if L1 > 0:
                c2.wait()
            jv = [jet_v[pl.ds(t * LANES, LANES)] for t in range(nblk)]

            def row_body(r, c):
                for t in range(nblk):
                    plsc.addupdate(buf_v.at[r, pl.ds(t * LANES, LANES)], jv[t])
                return c

            lax.fori_loop(0, Lv, row_body, 0, unroll=4)
            pltpu.sync_copy(buf_v, out_hbm.at[gb])
            return carry

        lax.fori_loop(0, nb, body, 0)

    return sc_k(x, jet, table)


def kernel(x, x_jet, table, W_jet):
    jet = _jet_proj(x_jet, W_jet)
    return _embed_add(x.astype(jnp.int32), jet, table)


# speedup vs baseline: 1.2139x; 1.2139x over previous
import functools

import jax
import jax.numpy as jnp
from jax import lax
from jax.experimental import pallas as pl
from jax.experimental.pallas import tpu as pltpu
from jax.experimental.pallas import tpu_sc as plsc

_NBUF = 4


def _jet_proj(x_jet, W_jet):
    Bv = x_jet.shape[0]
    EMBv = W_jet.shape[0]

    def body(xj_ref, w_ref, out_ref):
        out_ref[:] = lax.dot_general(
            xj_ref[:], w_ref[:],
            dimension_numbers=(((1,), (1,)), ((), ())),
            preferred_element_type=jnp.float32)

    return pl.pallas_call(
        body,
        out_shape=jax.ShapeDtypeStruct((Bv, EMBv), jnp.float32),
    )(x_jet, W_jet)


def _embed_add(x, jet, table):
    Bv, Lv = x.shape
    EMBv = table.shape[1]
    LANES = 16
    nblk = EMBv // LANES

    mesh = plsc.VectorSubcoreMesh(core_axis_name="c", subcore_axis_name="s")
    NC, NS = mesh.num_cores, mesh.num_subcores
    NW = NC * NS
    nb = Bv // NW

    L0 = min(128, Lv)
    L1 = Lv - L0

    @functools.partial(
        pl.kernel,
        out_type=jax.ShapeDtypeStruct((Bv, Lv, EMBv), jnp.float32),
        mesh=mesh,
        scratch_types=[
            pltpu.VMEM((nb, Lv), jnp.int32),
            pltpu.VMEM((nb, EMBv), jnp.float32),
            pltpu.VMEM((_NBUF, Lv, EMBv), jnp.float32),
        ] + [pltpu.SemaphoreType.DMA] * (2 * _NBUF),
        compiler_params=pltpu.CompilerParams(use_tc_tiling_on_sc=False),
    )
    def sc_k(x_hbm, jet_hbm, table_hbm, out_hbm, idx_v, jet_v, buf_v, *sems):
        sem_g = sems[:_NBUF]
        sem_o = sems[_NBUF:]
        wid = lax.axis_index("s") * NC + lax.axis_index("c")
        base_b = wid * nb

        pltpu.sync_copy(x_hbm.at[pl.ds(base_b, nb)], idx_v)
        pltpu.sync_copy(jet_hbm.at[pl.ds(base_b, nb)], jet_v)

        def issue_gather(jj, slot):
            pltpu.async_copy(
                table_hbm.at[idx_v.at[jj, pl.ds(0, L0)]],
                buf_v.at[slot, pl.ds(0, L0)], sem_g[slot])
            if L1 > 0:
                pltpu.async_copy(
                    table_hbm.at[idx_v.at[jj, pl.ds(L0, L1)]],
                    buf_v.at[slot, pl.ds(L0, L1)], sem_g[slot])

        def drain_gather(jj, slot):
            pltpu.make_async_copy(
                table_hbm.at[idx_v.at[jj, pl.ds(0, L0)]],
                buf_v.at[slot, pl.ds(0, L0)], sem_g[slot]).wait()
            if L1 > 0:
                pltpu.make_async_copy(
                    table_hbm.at[idx_v.at[jj, pl.ds(L0, L1)]],
                    buf_v.at[slot, pl.ds(L0, L1)], sem_g[slot]).wait()

        def drain_out(gb, slot):
            pltpu.make_async_copy(
                buf_v.at[slot], out_hbm.at[gb], sem_o[slot]).wait()

        issue_gather(0, 0)
        issue_gather(1, 1)

        def outer(jo, carry):
            for k in range(_NBUF):
                jj = _NBUF * jo + k
                s2 = (k + 2) % _NBUF

                @pl.when(jj + 2 < nb)
                def _():
                    @pl.when(jj >= 2)
                    def _():
                        drain_out(base_b + jj - 2, s2)
                    issue_gather(jj + 2, s2)

                drain_gather(jj, k)

                jv = [jet_v[jj, pl.ds(t * LANES, LANES)] for t in range(nblk)]

                def row_body(r, c):
                    for t in range(nblk):
                        plsc.addupdate(buf_v.at[k, r, pl.ds(t * LANES, LANES)],
                                       jv[t])
                    return c

                lax.fori_loop(0, Lv, row_body, 0, unroll=4)
                pltpu.async_copy(buf_v.at[k], out_hbm.at[base_b + jj],
                                 sem_o[k])
            return carry

        lax.fori_loop(0, nb // _NBUF, outer, 0)

        for k in range(_NBUF):
            drain_out(base_b + nb - _NBUF + k, (nb - _NBUF + k) % _NBUF)

    return sc_k(x, jet, table)


def kernel(x, x_jet, table, W_jet):
    jet = _jet_proj(x_jet, W_jet)
    return _embed_add(x.astype(jnp.int32), jet, table)
